# baseline (speedup 1.0000x reference)
import functools

import jax
import jax.numpy as jnp
from jax import lax
from jax.experimental import pallas as pl
from jax.experimental.pallas import tpu as pltpu
from jax.experimental.pallas import tpu_sc as plsc

K = 32
C = 128
GD = 384
S_CHUNK = 512
N_SRC_PAD = 51200
N_CHUNKS = N_SRC_PAD // S_CHUNK
N_DST_PAD = 10112
DBLK = 128
ABLK = 64
PBLK = 256

_SC_NC = 2
_SC_NS = 16
_SC_NW = _SC_NC * _SC_NS


def _silu(x):
    return x * (1.0 / (1.0 + jnp.exp(-x)))


def _proj_body(sf_ref, sp_ref, wcat_ref, wp_ref, g_ref):
    sf = sf_ref[...]
    g_ref[:, 0:2 * C] = jnp.dot(sf, wcat_ref[...],
                                preferred_element_type=jnp.float32)
    sp = sp_ref[...]
    wp = wp_ref[...]
    acc = jnp.zeros((PBLK, C), jnp.float32)
    for d in range(3):
        acc = acc + sp[:, d:d + 1] * wp[d:d + 1, :]
    g_ref[:, 2 * C:GD] = acc


def _knn_body(tp_ref, spt_ref, o_ref, bd_ref, bi_ref, dc_ref, ic_ref):
    c = pl.program_id(1)

    @pl.when(c == 0)
    def _():
        bd_ref[...] = jnp.full((DBLK, K), 1e30, jnp.float32)
        bi_ref[...] = jnp.full((DBLK, K), 1e9, jnp.float32)

    tb = tp_ref[...]
    sp = spt_ref[...]
    d0 = tb[:, 0:1] - sp[0:1, :]
    d1 = tb[:, 1:2] - sp[1:2, :]
    d2 = tb[:, 2:3] - sp[2:3, :]
    d2c = (d0 * d0 + d1 * d1) + d2 * d2

    theta = bd_ref[:, K - 1:K]
    cnt = jnp.sum((d2c < theta).astype(jnp.int32), axis=1, keepdims=True)
    h = jnp.minimum(jnp.max(cnt), K)

    @pl.when(h > 0)
    def _():
        base = (c * S_CHUNK).astype(jnp.float32)
        dc_ref[...] = d2c
        ic_ref[...] = lax.broadcasted_iota(
            jnp.int32, (DBLK, S_CHUNK), 1).astype(jnp.float32) + base
        lane = lax.broadcasted_iota(jnp.int32, (DBLK, K), 1)

        def body(t, carry):
            dcv = dc_ref[...]
            icv = ic_ref[...]
            rowmin = jnp.min(dcv, axis=1, keepdims=True)
            seli = jnp.min(jnp.where(dcv == rowmin, icv, 1e9),
                           axis=1, keepdims=True)
            dc_ref[...] = jnp.where(icv == seli, 1e30, dcv)
            bd = bd_ref[...]
            bi = bi_ref[...]
            pos = jnp.sum((bd <= rowmin).astype(jnp.int32), axis=1,
                          keepdims=True)
            bd_sh = jnp.concatenate([bd[:, :1], bd[:, :K - 1]], axis=1)
            bi_sh = jnp.concatenate([bi[:, :1], bi[:, :K - 1]], axis=1)
            bd_ref[...] = jnp.where(
                lane < pos, bd, jnp.where(lane == pos, rowmin, bd_sh))
            bi_ref[...] = jnp.where(
                lane < pos, bi, jnp.where(lane == pos, seli, bi_sh))
            return carry

        lax.fori_loop(0, h, body, 0)

    @pl.when(c == N_CHUNKS - 1)
    def _():
        o_ref[...] = bi_ref[...].astype(jnp.int32)


def _sc_gather(table, idxf):
    b = idxf.shape[0]
    per_w = b // _SC_NW
    chunk = 128
    n_ch = per_w // chunk
    mesh = plsc.VectorSubcoreMesh(core_axis_name="c", subcore_axis_name="s")

    @functools.partial(
        pl.kernel, mesh=mesh,
        out_type=jax.ShapeDtypeStruct((b, GD), jnp.float32),
        scratch_types=[
            pltpu.VMEM((chunk,), jnp.int32),
            pltpu.VMEM((chunk, GD), jnp.float32),
            pltpu.SemaphoreType.DMA,
        ],
    )
    def gk(table_hbm, idx_hbm, out_hbm, idx_v, rows_v, sem):
        wid = lax.axis_index("s") * _SC_NC + lax.axis_index("c")
        base = wid * per_w

        def body(ci, carry):
            start = pl.multiple_of(base + ci * chunk, chunk)
            pltpu.sync_copy(idx_hbm.at[pl.ds(start, chunk)], idx_v)
            pltpu.async_copy(table_hbm.at[idx_v], rows_v, sem).wait()
            pltpu.sync_copy(rows_v, out_hbm.at[pl.ds(start, chunk)])
            return carry

        lax.fori_loop(0, n_ch, body, 0)

    return gk(table, idxf)


def _attn_body(e_ref, tf_ref, tp_ref, wd_ref, wp_ref, bp_ref, wa_ref,
               ba_ref, o_ref):
    eb = e_ref[...]
    xl = eb[:, :, 0:C]
    asr = eb[:, :, C:2 * C]
    psr = eb[:, :, 2 * C:GD]
    ad = jnp.dot(tf_ref[...], wd_ref[...],
                 preferred_element_type=jnp.float32)
    tp = tp_ref[...]
    wp = wp_ref[...]
    pd = jnp.broadcast_to(bp_ref[...], (ABLK, C))
    for d in range(3):
        pd = pd + tp[:, d:d + 1] * wp[d:d + 1, :]
    delta = _silu(pd[:, None, :] - psr)
    a_in = ad[:, None, :] - asr + delta
    z = jnp.dot(a_in.reshape(ABLK * K, C), wa_ref[...],
                preferred_element_type=jnp.float32) + ba_ref[...]
    a3 = _silu(z).reshape(ABLK, K, C)
    m = a3[:, 0, :]
    for k in range(1, K):
        m = jnp.maximum(m, a3[:, k, :])
    s = jnp.zeros((ABLK, C), jnp.float32)
    acc = jnp.zeros((ABLK, C), jnp.float32)
    for k in range(K):
        p = jnp.exp(a3[:, k, :] - m)
        s = s + p
        acc = acc + p * (xl[:, k, :] + delta[:, k, :])
    o_ref[...] = acc / s


def kernel(source_pos, target_pos, source_feat, target_feat,
           Wp, bp, Wa, ba, Wl, Ws, Wd):
    f32 = jnp.float32
    n_src = source_pos.shape[0]
    n_dst = target_pos.shape[0]

    sf = jnp.zeros((N_SRC_PAD, C), f32).at[:n_src].set(source_feat)
    sp = jnp.zeros((N_SRC_PAD, 8), f32)
    sp = sp.at[:n_src, :3].set(source_pos)
    sp = sp.at[n_src:, :3].set(100.0)
    tpp = jnp.zeros((N_DST_PAD, 8), f32).at[:n_dst, :3].set(target_pos)
    tf = jnp.zeros((N_DST_PAD, C), f32).at[:n_dst].set(target_feat)
    wcat = jnp.concatenate([Wl, Ws], axis=1)
    wp8 = jnp.zeros((8, C), f32).at[:3].set(Wp)
    bp2 = bp.reshape(1, C)
    ba2 = ba.reshape(1, C)
    spt = sp.T

    g = pl.pallas_call(
        _proj_body,
        grid=(N_SRC_PAD // PBLK,),
        in_specs=[
            pl.BlockSpec((PBLK, C), lambda i: (i, 0)),
            pl.BlockSpec((PBLK, 8), lambda i: (i, 0)),
            pl.BlockSpec((C, 2 * C), lambda i: (0, 0)),
            pl.BlockSpec((8, C), lambda i: (0, 0)),
        ],
        out_specs=pl.BlockSpec((PBLK, GD), lambda i: (i, 0)),
        out_shape=jax.ShapeDtypeStruct((N_SRC_PAD, GD), f32),
    )(sf, sp, wcat, wp8)

    idx = pl.pallas_call(
        _knn_body,
        grid=(N_DST_PAD // DBLK, N_CHUNKS),
        in_specs=[
            pl.BlockSpec((DBLK, 8), lambda j, c: (j, 0)),
            pl.BlockSpec((8, S_CHUNK), lambda j, c: (0, c)),
        ],
        out_specs=pl.BlockSpec((DBLK, K), lambda j, c: (j, 0)),
        out_shape=jax.ShapeDtypeStruct((N_DST_PAD, K), jnp.int32),
        scratch_shapes=[
            pltpu.VMEM((DBLK, K), f32),
            pltpu.VMEM((DBLK, K), f32),
            pltpu.VMEM((DBLK, S_CHUNK), f32),
            pltpu.VMEM((DBLK, S_CHUNK), f32),
        ],
    )(tpp, spt)

    e = _sc_gather(g, idx.reshape(-1))
    e3 = e.reshape(N_DST_PAD, K, GD)

    out = pl.pallas_call(
        _attn_body,
        grid=(N_DST_PAD // ABLK,),
        in_specs=[
            pl.BlockSpec((ABLK, K, GD), lambda i: (i, 0, 0)),
            pl.BlockSpec((ABLK, C), lambda i: (i, 0)),
            pl.BlockSpec((ABLK, 8), lambda i: (i, 0)),
            pl.BlockSpec((C, C), lambda i: (0, 0)),
            pl.BlockSpec((8, C), lambda i: (0, 0)),
            pl.BlockSpec((1, C), lambda i: (0, 0)),
            pl.BlockSpec((C, C), lambda i: (0, 0)),
            pl.BlockSpec((1, C), lambda i: (0, 0)),
        ],
        out_specs=pl.BlockSpec((ABLK, C), lambda i: (i, 0)),
        out_shape=jax.ShapeDtypeStruct((N_DST_PAD, C), f32),
    )(e3, tf, tpp, Wd, wp8, bp2, Wa, ba2)

    return out[:n_dst]


# speedup vs baseline: 1.2460x; 1.2460x over previous
import functools

import jax
import jax.numpy as jnp
from jax import lax
from jax.experimental import pallas as pl
from jax.experimental.pallas import tpu as pltpu
from jax.experimental.pallas import tpu_sc as plsc

K = 32
C = 128
GD = 384
S_CHUNK = 2048
HW = 1024
N_SRC_PAD = 51200
N_CHUNKS = N_SRC_PAD // S_CHUNK
N_DST_PAD = 10112
DBLK = 128
ABLK = 64
PBLK = 256

_SC_NC = 2
_SC_NS = 16
_SC_NW = _SC_NC * _SC_NS


def _silu(x):
    return x * (1.0 / (1.0 + jnp.exp(-x)))


def _proj_body(sf_ref, sp_ref, wcat_ref, wp_ref, g_ref):
    sf = sf_ref[...]
    g_ref[:, 0:2 * C] = jnp.dot(sf, wcat_ref[...],
                                preferred_element_type=jnp.float32)
    sp = sp_ref[...]
    wp = wp_ref[...]
    acc = jnp.zeros((PBLK, C), jnp.float32)
    for d in range(3):
        acc = acc + sp[:, d:d + 1] * wp[d:d + 1, :]
    g_ref[:, 2 * C:GD] = acc


def _knn_body(tp_ref, spt_ref, o_ref, bd_ref, bi_ref, dc_ref, ic_ref):
    c = pl.program_id(1)

    @pl.when(c == 0)
    def _():
        bd_ref[...] = jnp.full((DBLK, K), 1e30, jnp.float32)
        bi_ref[...] = jnp.full((DBLK, K), 1e9, jnp.float32)

    tb = tp_ref[...]
    sp = spt_ref[...]
    d0 = tb[:, 0:1] - sp[0:1, :]
    d1 = tb[:, 1:2] - sp[1:2, :]
    d2 = tb[:, 2:3] - sp[2:3, :]
    d2c = (d0 * d0 + d1 * d1) + d2 * d2

    lane = lax.broadcasted_iota(jnp.int32, (DBLK, K), 1)
    for half in range(S_CHUNK // HW):
        dh = d2c[:, half * HW:(half + 1) * HW]
        theta = bd_ref[:, K - 1:K]
        cnt = jnp.sum((dh < theta).astype(jnp.int32), axis=1, keepdims=True)
        h = jnp.minimum(jnp.max(cnt), K)

        @pl.when(h > 0)
        def _(dh=dh, half=half, h=h):
            base = (c * S_CHUNK + half * HW).astype(jnp.float32)
            dc_ref[...] = dh
            ic_ref[...] = lax.broadcasted_iota(
                jnp.int32, (DBLK, HW), 1).astype(jnp.float32) + base

            def body(t, carry):
                dcv = dc_ref[...]
                icv = ic_ref[...]
                rowmin = jnp.min(dcv, axis=1, keepdims=True)
                seli = jnp.min(jnp.where(dcv == rowmin, icv, 1e9),
                               axis=1, keepdims=True)
                dc_ref[...] = jnp.where(icv == seli, 1e30, dcv)
                bd = bd_ref[...]
                bi = bi_ref[...]
                pos = jnp.sum((bd <= rowmin).astype(jnp.int32), axis=1,
                              keepdims=True)
                bd_sh = jnp.concatenate([bd[:, :1], bd[:, :K - 1]], axis=1)
                bi_sh = jnp.concatenate([bi[:, :1], bi[:, :K - 1]], axis=1)
                bd_ref[...] = jnp.where(
                    lane < pos, bd, jnp.where(lane == pos, rowmin, bd_sh))
                bi_ref[...] = jnp.where(
                    lane < pos, bi, jnp.where(lane == pos, seli, bi_sh))
                return carry

            lax.fori_loop(0, h, body, 0)

    @pl.when(c == N_CHUNKS - 1)
    def _():
        o_ref[...] = bi_ref[...].astype(jnp.int32)


def _sc_gather(table, idxf):
    b = idxf.shape[0]
    per_w = b // _SC_NW
    chunk = 128
    n_ch = per_w // chunk
    mesh = plsc.VectorSubcoreMesh(core_axis_name="c", subcore_axis_name="s")

    @functools.partial(
        pl.kernel, mesh=mesh,
        out_type=jax.ShapeDtypeStruct((b, GD), jnp.float32),
        scratch_types=[
            pltpu.VMEM((chunk,), jnp.int32),
            pltpu.VMEM((chunk, GD), jnp.float32),
            pltpu.SemaphoreType.DMA,
        ],
    )
    def gk(table_hbm, idx_hbm, out_hbm, idx_v, rows_v, sem):
        wid = lax.axis_index("s") * _SC_NC + lax.axis_index("c")
        base = wid * per_w

        def body(ci, carry):
            start = pl.multiple_of(base + ci * chunk, chunk)
            pltpu.sync_copy(idx_hbm.at[pl.ds(start, chunk)], idx_v)
            pltpu.async_copy(table_hbm.at[idx_v], rows_v, sem).wait()
            pltpu.sync_copy(rows_v, out_hbm.at[pl.ds(start, chunk)])
            return carry

        lax.fori_loop(0, n_ch, body, 0)

    return gk(table, idxf)


def _attn_body(e_ref, tf_ref, tp_ref, wd_ref, wp_ref, bp_ref, wa_ref,
               ba_ref, o_ref):
    eb = e_ref[...]
    xl = eb[:, :, 0:C]
    asr = eb[:, :, C:2 * C]
    psr = eb[:, :, 2 * C:GD]
    ad = jnp.dot(tf_ref[...], wd_ref[...],
                 preferred_element_type=jnp.float32)
    tp = tp_ref[...]
    wp = wp_ref[...]
    pd = jnp.broadcast_to(bp_ref[...], (ABLK, C))
    for d in range(3):
        pd = pd + tp[:, d:d + 1] * wp[d:d + 1, :]
    delta = _silu(pd[:, None, :] - psr)
    a_in = ad[:, None, :] - asr + delta
    z = jnp.dot(a_in.reshape(ABLK * K, C), wa_ref[...],
                preferred_element_type=jnp.float32) + ba_ref[...]
    a3 = _silu(z).reshape(ABLK, K, C)
    m = a3[:, 0, :]
    for k in range(1, K):
        m = jnp.maximum(m, a3[:, k, :])
    s = jnp.zeros((ABLK, C), jnp.float32)
    acc = jnp.zeros((ABLK, C), jnp.float32)
    for k in range(K):
        p = jnp.exp(a3[:, k, :] - m)
        s = s + p
        acc = acc + p * (xl[:, k, :] + delta[:, k, :])
    o_ref[...] = acc / s


def kernel(source_pos, target_pos, source_feat, target_feat,
           Wp, bp, Wa, ba, Wl, Ws, Wd):
    f32 = jnp.float32
    n_src = source_pos.shape[0]
    n_dst = target_pos.shape[0]

    sf = jnp.zeros((N_SRC_PAD, C), f32).at[:n_src].set(source_feat)
    sp = jnp.zeros((N_SRC_PAD, 8), f32)
    sp = sp.at[:n_src, :3].set(source_pos)
    sp = sp.at[n_src:, :3].set(100.0)
    tpp = jnp.zeros((N_DST_PAD, 8), f32).at[:n_dst, :3].set(target_pos)
    tf = jnp.zeros((N_DST_PAD, C), f32).at[:n_dst].set(target_feat)
    wcat = jnp.concatenate([Wl, Ws], axis=1)
    wp8 = jnp.zeros((8, C), f32).at[:3].set(Wp)
    bp2 = bp.reshape(1, C)
    ba2 = ba.reshape(1, C)
    spt = sp.T

    g = pl.pallas_call(
        _proj_body,
        grid=(N_SRC_PAD // PBLK,),
        in_specs=[
            pl.BlockSpec((PBLK, C), lambda i: (i, 0)),
            pl.BlockSpec((PBLK, 8), lambda i: (i, 0)),
            pl.BlockSpec((C, 2 * C), lambda i: (0, 0)),
            pl.BlockSpec((8, C), lambda i: (0, 0)),
        ],
        out_specs=pl.BlockSpec((PBLK, GD), lambda i: (i, 0)),
        out_shape=jax.ShapeDtypeStruct((N_SRC_PAD, GD), f32),
    )(sf, sp, wcat, wp8)

    idx = pl.pallas_call(
        _knn_body,
        grid=(N_DST_PAD // DBLK, N_CHUNKS),
        in_specs=[
            pl.BlockSpec((DBLK, 8), lambda j, c: (j, 0)),
            pl.BlockSpec((8, S_CHUNK), lambda j, c: (0, c)),
        ],
        out_specs=pl.BlockSpec((DBLK, K), lambda j, c: (j, 0)),
        out_shape=jax.ShapeDtypeStruct((N_DST_PAD, K), jnp.int32),
        scratch_shapes=[
            pltpu.VMEM((DBLK, K), f32),
            pltpu.VMEM((DBLK, K), f32),
            pltpu.VMEM((DBLK, HW), f32),
            pltpu.VMEM((DBLK, HW), f32),
        ],
    )(tpp, spt)

    e = _sc_gather(g, idx.reshape(-1))
    e3 = e.reshape(N_DST_PAD, K, GD)

    out = pl.pallas_call(
        _attn_body,
        grid=(N_DST_PAD // ABLK,),
        in_specs=[
            pl.BlockSpec((ABLK, K, GD), lambda i: (i, 0, 0)),
            pl.BlockSpec((ABLK, C), lambda i: (i, 0)),
            pl.BlockSpec((ABLK, 8), lambda i: (i, 0)),
            pl.BlockSpec((C, C), lambda i: (0, 0)),
            pl.BlockSpec((8, C), lambda i: (0, 0)),
            pl.BlockSpec((1, C), lambda i: (0, 0)),
            pl.BlockSpec((C, C), lambda i: (0, 0)),
            pl.BlockSpec((1, C), lambda i: (0, 0)),
        ],
        out_specs=pl.BlockSpec((ABLK, C), lambda i: (i, 0)),
        out_shape=jax.ShapeDtypeStruct((N_DST_PAD, C), f32),
    )(e3, tf, tpp, Wd, wp8, bp2, Wa, ba2)

    return out[:n_dst]
